# Initial kernel scaffold; baseline (speedup 1.0000x reference)
import jax
import jax.numpy as jnp
from jax.experimental import pallas as pl


def kernel(x, messages, dst, W, b):
    raise NotImplementedError("write your pallas kernel here")


# speedup vs baseline: 3.7165x; 3.7165x over previous
import functools

import jax
import jax.numpy as jnp
from jax import lax
from jax.experimental import pallas as pl
from jax.experimental.pallas import tpu as pltpu
from jax.experimental.pallas import tpu_sc as plsc

NN = 10000
EE = 320000
DD = 128

NC = 2
NS = 16
LL = 16
NW = NC * NS
EPW = EE // NW
K = 80
C = EPW // K
RPT = NN // NS


def _tc_atty(x, W, b):
    def body(x_ref, w_ref, b_ref, o_ref):
        o_ref[...] = jnp.tanh(
            jnp.dot(x_ref[...], w_ref[...], preferred_element_type=jnp.float32)
            + b_ref[...]
        )

    return pl.pallas_call(
        body,
        grid=(10,),
        in_specs=[
            pl.BlockSpec((NN // 10, DD), lambda i: (i, 0)),
            pl.BlockSpec((DD, DD), lambda i: (0, 0)),
            pl.BlockSpec((1, DD), lambda i: (0, 0)),
        ],
        out_specs=pl.BlockSpec((NN // 10, DD), lambda i: (i, 0)),
        out_shape=jax.ShapeDtypeStruct((NN, DD), jnp.float32),
    )(x, W, b.reshape(1, DD))


def _tc_add(p0, p1):
    def body(a_ref, b_ref, o_ref):
        o_ref[...] = a_ref[...] + b_ref[...]

    return pl.pallas_call(
        body,
        grid=(10,),
        in_specs=[
            pl.BlockSpec((NN // 10, DD), lambda i: (i, 0)),
            pl.BlockSpec((NN // 10, DD), lambda i: (i, 0)),
        ],
        out_specs=pl.BlockSpec((NN // 10, DD), lambda i: (i, 0)),
        out_shape=jax.ShapeDtypeStruct((NN, DD), jnp.float32),
    )(p0, p1)


@functools.partial(
    pl.kernel,
    out_type=jax.ShapeDtypeStruct((2 * NN, DD), jnp.float32),
    mesh=plsc.VectorSubcoreMesh(core_axis_name="c", subcore_axis_name="s"),
    scratch_types=[
        pltpu.VMEM((2, K), jnp.int32),
        pltpu.VMEM((2, K, DD), jnp.float32),
        pltpu.VMEM((2, K, DD), jnp.float32),
        pltpu.VMEM_SHARED((NN, DD), jnp.float32),
        pltpu.SemaphoreType.DMA,
        pltpu.SemaphoreType.DMA,
        pltpu.SemaphoreType.DMA,
        pltpu.SemaphoreType.DMA,
    ],
)
def _sc_attend(atty_hbm, dst2_hbm, msg_hbm, out_hbm,
               idx_all, gbuf, mbuf, acc, sg0, sg1, sm0, sm1):
    c = lax.axis_index("c")
    s = lax.axis_index("s")
    wid = s * NC + c
    ebase = wid * EPW

    zv = jnp.zeros((LL,), jnp.float32)

    def zrow(i, carry):
        for j in range(DD // LL):
            gbuf[0, i, pl.ds(j * LL, LL)] = zv
        return carry

    lax.fori_loop(0, K, zrow, 0)

    NBLK = NN // K

    def zacc(q, carry):
        blk = s + q * NS

        @pl.when(blk < NBLK)
        def _():
            pltpu.sync_copy(gbuf.at[0], acc.at[pl.ds(blk * K, K)])

        return carry

    lax.fori_loop(0, (NBLK + NS - 1) // NS, zacc, 0)
    plsc.subcore_barrier()

    sgs = (sg0, sg1)
    sms = (sm0, sm1)

    def start(ci, slot):
        pltpu.sync_copy(dst2_hbm.at[wid, ci], idx_all.at[slot])
        pltpu.async_copy(atty_hbm.at[idx_all.at[slot]], gbuf.at[slot],
                         sgs[slot])
        pltpu.async_copy(msg_hbm.at[pl.ds(ebase + ci * K, K)],
                         mbuf.at[slot], sms[slot])

    def wait(slot):
        pltpu.make_async_copy(atty_hbm.at[pl.ds(0, K)], gbuf.at[slot],
                              sgs[slot]).wait()
        pltpu.make_async_copy(msg_hbm.at[pl.ds(0, K)], mbuf.at[slot],
                              sms[slot]).wait()

    def compute(slot):
        def body(i, carry):
            accv = jnp.zeros((LL,), jnp.float32)
            ms = []
            for j in range(DD // LL):
                gj = gbuf[slot, i, pl.ds(j * LL, LL)]
                mj = mbuf[slot, i, pl.ds(j * LL, LL)]
                ms.append(mj)
                accv = accv + gj * mj
            lanes = lax.iota(jnp.int32, LL)
            for sh in (8, 4, 2, 1):
                accv = accv + accv.at[lanes ^ sh].get(
                    mode="promise_in_bounds")
            sig = 1.0 / (1.0 + jnp.exp(-accv))
            for j in range(DD // LL):
                mbuf[slot, i, pl.ds(j * LL, LL)] = sig * ms[j]
            return carry

        lax.fori_loop(0, K, body, 0)

    def scatter(ci, slot):
        pltpu.sync_copy(mbuf.at[slot], acc.at[idx_all.at[slot]], add=True)

    start(0, 0)

    def pair(p, carry):
        ci0 = p * 2
        start(ci0 + 1, 1)
        wait(0)
        compute(0)
        scatter(ci0, 0)

        @pl.when(ci0 + 2 < C)
        def _():
            start(ci0 + 2, 0)

        wait(1)
        compute(1)
        scatter(ci0 + 1, 1)
        return carry

    lax.fori_loop(0, C // 2, pair, 0)
    wait(0)
    compute(0)
    scatter(C - 1, 0)

    plsc.subcore_barrier()

    def flush(q, carry):
        blk = s + q * NS

        @pl.when(blk < NBLK)
        def _():
            pltpu.sync_copy(acc.at[pl.ds(blk * K, K)],
                            out_hbm.at[pl.ds(c * NN + blk * K, K)])

        return carry

    lax.fori_loop(0, (NBLK + NS - 1) // NS, flush, 0)


def kernel(x, messages, dst, W, b):
    atty = _tc_atty(x, W, b)
    dst2 = dst.reshape(NW, C, K)
    partial = _sc_attend(atty, dst2, messages)
    return _tc_add(partial[:NN], partial[NN:])
